# baseline (speedup 1.0000x reference)
import jax
import jax.numpy as jnp
from jax import lax
from jax.experimental import pallas as pl
from jax.experimental.pallas import tpu as pltpu
from jax.experimental.pallas import tpu_sc as plsc

_N = 10000
_E = 320000
_D = 128
_NPAD = 10240
_RPT = _NPAD // 16
_CH = 128
_EC = _E // _CH

_MESH = dict(core_axis_name="c", subcore_axis_name="s",
             num_cores=2, num_subcores=16)


_K1_CNT = 80


def _deg_body(col2_hbm, degp_hbm, degacc, stage, onesb, idxslab, sems):
    c = lax.axis_index("c")
    s = lax.axis_index("s")
    w = c * 16 + s

    def _z(i, _):
        stage[pl.ds(i * 16, 16)] = jnp.zeros((16,), jnp.float32)
        return 0

    lax.fori_loop(0, _RPT // 16, _z, 0)

    def _o(i, _):
        onesb[pl.ds(i * 16, 16)] = jnp.ones((16,), jnp.float32)
        return 0

    lax.fori_loop(0, _CH // 16, _o, 0)
    pltpu.sync_copy(stage, degacc.at[pl.ds(s * _RPT, _RPT)])

    @pl.when(w < 31)
    def _():
        pltpu.sync_copy(col2_hbm.at[pl.ds(w * _K1_CNT, _K1_CNT)], idxslab)

    @pl.when(w == 31)
    def _():
        pltpu.sync_copy(col2_hbm.at[pl.ds(31 * _K1_CNT, _EC - 31 * _K1_CNT)],
                        idxslab.at[pl.ds(0, _EC - 31 * _K1_CNT)])

    plsc.subcore_barrier()

    def _fire(j, _):
        pltpu.async_copy(onesb, degacc.at[idxslab.at[j]], sems, add=True)
        return 0

    def _drain(j, _):
        pltpu.make_async_copy(onesb, degacc.at[idxslab.at[0]], sems).wait()
        return 0

    @pl.when(w < 31)
    def _():
        lax.fori_loop(0, _K1_CNT, _fire, 0)
        lax.fori_loop(0, _K1_CNT, _drain, 0)

    @pl.when(w == 31)
    def _():
        lax.fori_loop(0, _EC - 31 * _K1_CNT, _fire, 0)
        lax.fori_loop(0, _EC - 31 * _K1_CNT, _drain, 0)

    plsc.subcore_barrier()
    pltpu.sync_copy(degacc.at[pl.ds(s * _RPT, _RPT)], stage)
    pltpu.sync_copy(stage, degp_hbm.at[c, pl.ds(s * _RPT, _RPT)])


def _make_deg():
    return pl.kernel(
        _deg_body,
        out_type=jax.ShapeDtypeStruct((2, _NPAD), jnp.float32),
        mesh=plsc.VectorSubcoreMesh(**_MESH),
        scratch_types=[
            pltpu.VMEM_SHARED((_NPAD,), jnp.float32),
            pltpu.VMEM((_RPT,), jnp.float32),
            pltpu.VMEM((_CH,), jnp.float32),
            pltpu.VMEM((_K1_CNT, _CH), jnp.int32),
            pltpu.SemaphoreType.DMA,
        ],
    )


_BLK = 1024


def _prescale_body(x_ref, m_ref, degt_ref, yp_ref, mp_ref, dinv_ref):
    dsum = degt_ref[:, 0:1] + degt_ref[:, 1:2]
    dv = jnp.where(dsum > 0, lax.rsqrt(dsum), 0.0)
    mm = m_ref[...]
    yp_ref[...] = x_ref[...] * mm * dv
    mp_ref[...] = mm * dv
    dinv_ref[...] = dv


def _make_prescale():
    return pl.pallas_call(
        _prescale_body,
        grid=(_NPAD // _BLK,),
        in_specs=[
            pl.BlockSpec((_BLK, _D), lambda i: (i, 0)),
            pl.BlockSpec((_BLK, _D), lambda i: (i, 0)),
            pl.BlockSpec((_BLK, 2), lambda i: (i, 0)),
        ],
        out_specs=[
            pl.BlockSpec((_BLK, _D), lambda i: (i, 0)),
            pl.BlockSpec((_BLK, _D), lambda i: (i, 0)),
            pl.BlockSpec((_BLK, 1), lambda i: (i, 0)),
        ],
        out_shape=[
            jax.ShapeDtypeStruct((_NPAD, _D), jnp.float32),
            jax.ShapeDtypeStruct((_NPAD, _D), jnp.float32),
            jax.ShapeDtypeStruct((_NPAD, 1), jnp.float32),
        ],
    )


_K3_CNT = _EC // 16


def _agg_body(yp_hbm, mp_hbm, dinv_hbm, row1_hbm, col1_hbm,
              t2_hbm, t3_hbm, t1p_hbm,
              acc, t1acc, cb0, cb1, cb2, cb3, rb0, rb1, rb2, rb3,
              db0, db1, vb0, vb1,
              si0, si1, si2, si3, sg0, sg1, ss0, ss1, sv0, sv1, st0, st1):
    c = lax.axis_index("c")
    s = lax.axis_index("s")
    colb = (cb0, cb1, cb2, cb3)
    rowb = (rb0, rb1, rb2, rb3)
    datab = (db0, db1)
    valsb = (vb0, vb1)
    semi = (si0, si1, si2, si3)
    semg = (sg0, sg1)
    sems = (ss0, ss1)
    semv = (sv0, sv1)
    semt = (st0, st1)

    def _zd(t, _):
        db0[t // 8, pl.ds((t % 8) * 16, 16)] = jnp.zeros((16,), jnp.float32)
        return 0

    lax.fori_loop(0, _CH * (_D // 16), _zd, 0)

    def _zv(i, _):
        vb0[pl.ds(i * 16, 16)] = jnp.zeros((16,), jnp.float32)
        return 0

    lax.fori_loop(0, _CH // 16, _zv, 0)

    row0 = s * _RPT
    for q in range(_RPT // _CH):
        pltpu.sync_copy(db0, acc.at[pl.ds(row0 + q * _CH, _CH)])
        pltpu.sync_copy(vb0, t1acc.at[pl.ds(row0 + q * _CH, _CH)])
    plsc.subcore_barrier()

    ebase = s * _K3_CNT

    def _issue_i(j, a):
        off = (ebase + j) * _CH
        pltpu.async_copy(col1_hbm.at[pl.ds(off, _CH)], colb[a], semi[a])
        pltpu.async_copy(row1_hbm.at[pl.ds(off, _CH)], rowb[a], semi[a])

    def _wait_i(j, a):
        off = (ebase + j) * _CH
        pltpu.make_async_copy(col1_hbm.at[pl.ds(off, _CH)], colb[a],
                              semi[a]).wait()
        pltpu.make_async_copy(row1_hbm.at[pl.ds(off, _CH)], rowb[a],
                              semi[a]).wait()

    def _run(table, with_t1, cbase, cnt):
        def issue_g(k, a, p):
            pltpu.async_copy(table.at[colb[a]], datab[p], semg[p])
            if with_t1:
                pltpu.async_copy(dinv_hbm.at[colb[a]], valsb[p], semv[p])

        def wait_g(a, p):
            pltpu.make_async_copy(table.at[colb[a]], datab[p], semg[p]).wait()
            if with_t1:
                pltpu.make_async_copy(dinv_hbm.at[colb[a]], valsb[p],
                                      semv[p]).wait()

        def issue_s(a, p):
            pltpu.async_copy(datab[p], acc.at[rowb[a]], sems[p], add=True)
            if with_t1:
                pltpu.async_copy(valsb[p], t1acc.at[rowb[a]], semt[p],
                                 add=True)

        def wait_s(a, p):
            pltpu.make_async_copy(datab[p], acc.at[rowb[a]], sems[p]).wait()
            if with_t1:
                pltpu.make_async_copy(valsb[p], t1acc.at[rowb[a]],
                                      semt[p]).wait()

        _issue_i(cbase + 0, 0)
        _issue_i(cbase + 1, 1)
        _issue_i(cbase + 2, 2)
        _wait_i(cbase + 0, 0)
        issue_g(0, 0, 0)
        _wait_i(cbase + 1, 1)
        issue_g(1, 1, 1)

        def _step(k, a, p):
            a2 = (a + 2) % 4
            wait_g(a2, p)
            issue_s(a2, p)
            wait_s(a2, p)
            _issue_i(cbase + k + 1, (a + 1) % 4)
            _wait_i(cbase + k, a)
            issue_g(k, a, p)

        def _quad(jj, _):
            k0 = 4 * jj + 2
            _step(k0, 2, 0)
            _step(k0 + 1, 3, 1)
            _step(k0 + 2, 0, 0)
            _step(k0 + 3, 1, 1)
            return 0

        nquad = (cnt - 2) // 4
        lax.fori_loop(0, nquad, _quad, 0)
        k0 = 4 * nquad + 2

        def _tail_step(k, a, p):
            a2 = (a + 2) % 4
            wait_g(a2, p)
            issue_s(a2, p)
            wait_s(a2, p)
            _wait_i(cbase + k, a)
            issue_g(k, a, p)

        if k0 < cnt:
            _issue_i(cbase + cnt - 1, (cnt - 1) % 4)
            _tail_step(k0, k0 % 4, 0)
            _tail_step(k0 + 1, (k0 + 1) % 4, 1)
        wait_g((cnt - 2) % 4, 0)
        issue_s((cnt - 2) % 4, 0)
        wait_s((cnt - 2) % 4, 0)
        wait_g((cnt - 1) % 4, 1)
        issue_s((cnt - 1) % 4, 1)
        wait_s((cnt - 1) % 4, 1)

    def _run_core(table, first_half_t1, cnt):
        half = cnt // 2
        _run(table, first_half_t1, 0, half)
        _run(table, not first_half_t1, half, half)

    @pl.when(jnp.logical_and(c == 0, s < 15))
    def _():
        _run_core(yp_hbm, True, _K3_CNT)

    @pl.when(jnp.logical_and(c == 0, s == 15))
    def _():
        _run_core(yp_hbm, True, _EC - 15 * _K3_CNT)

    @pl.when(jnp.logical_and(c == 1, s < 15))
    def _():
        _run_core(mp_hbm, False, _K3_CNT)

    @pl.when(jnp.logical_and(c == 1, s == 15))
    def _():
        _run_core(mp_hbm, False, _EC - 15 * _K3_CNT)

    plsc.subcore_barrier()
    for q in range(_RPT // _CH):
        r = row0 + q * _CH

        @pl.when(c == 0)
        def _out0():
            pltpu.sync_copy(acc.at[pl.ds(r, _CH)], db0)
            pltpu.sync_copy(db0, t2_hbm.at[pl.ds(r, _CH)])

        @pl.when(c == 1)
        def _out1():
            pltpu.sync_copy(acc.at[pl.ds(r, _CH)], db0)
            pltpu.sync_copy(db0, t3_hbm.at[pl.ds(r, _CH)])

        pltpu.sync_copy(t1acc.at[pl.ds(r, _CH)], vb0)
        pltpu.sync_copy(vb0, t1p_hbm.at[c, pl.ds(r, _CH)])


def _make_agg():
    return pl.kernel(
        _agg_body,
        out_type=(
            jax.ShapeDtypeStruct((_NPAD, _D), jnp.float32),
            jax.ShapeDtypeStruct((_NPAD, _D), jnp.float32),
            jax.ShapeDtypeStruct((2, _NPAD), jnp.float32),
        ),
        mesh=plsc.VectorSubcoreMesh(**_MESH),
        scratch_types=[
            pltpu.VMEM_SHARED((_NPAD, _D), jnp.float32),
            pltpu.VMEM_SHARED((_NPAD,), jnp.float32),
            pltpu.VMEM((_CH,), jnp.int32),
            pltpu.VMEM((_CH,), jnp.int32),
            pltpu.VMEM((_CH,), jnp.int32),
            pltpu.VMEM((_CH,), jnp.int32),
            pltpu.VMEM((_CH,), jnp.int32),
            pltpu.VMEM((_CH,), jnp.int32),
            pltpu.VMEM((_CH,), jnp.int32),
            pltpu.VMEM((_CH,), jnp.int32),
            pltpu.VMEM((_CH, _D), jnp.float32),
            pltpu.VMEM((_CH, _D), jnp.float32),
            pltpu.VMEM((_CH,), jnp.float32),
            pltpu.VMEM((_CH,), jnp.float32),
            pltpu.SemaphoreType.DMA,
            pltpu.SemaphoreType.DMA,
            pltpu.SemaphoreType.DMA,
            pltpu.SemaphoreType.DMA,
            pltpu.SemaphoreType.DMA,
            pltpu.SemaphoreType.DMA,
            pltpu.SemaphoreType.DMA,
            pltpu.SemaphoreType.DMA,
            pltpu.SemaphoreType.DMA,
            pltpu.SemaphoreType.DMA,
            pltpu.SemaphoreType.DMA,
            pltpu.SemaphoreType.DMA,
        ],
    )


def _final_body(t2_ref, t3_ref, t1p_ref, dinv_ref, w_ref, b_ref, o_ref):
    dv = dinv_ref[...]
    t1 = t1p_ref[:, 0:1] + t1p_ref[:, 1:2]
    t3 = t3_ref[...]
    safe = jnp.where(t3 != 0, t3, 1.0)
    nz = (t3 != 0) & (dv != 0)
    ratio = jnp.where(nz, dv * t1 * t2_ref[...] / safe, 0.0)
    o_ref[...] = lax.dot_general(
        ratio, w_ref[...], (((1,), (1,)), ((), ())),
        preferred_element_type=jnp.float32) + b_ref[...]


def _make_final():
    return pl.pallas_call(
        _final_body,
        grid=(_NPAD // _BLK,),
        in_specs=[
            pl.BlockSpec((_BLK, _D), lambda i: (i, 0)),
            pl.BlockSpec((_BLK, _D), lambda i: (i, 0)),
            pl.BlockSpec((_BLK, 2), lambda i: (i, 0)),
            pl.BlockSpec((_BLK, 1), lambda i: (i, 0)),
            pl.BlockSpec((_D, _D), lambda i: (0, 0)),
            pl.BlockSpec((1, _D), lambda i: (0, 0)),
        ],
        out_specs=pl.BlockSpec((_BLK, _D), lambda i: (i, 0)),
        out_shape=jax.ShapeDtypeStruct((_N, _D), jnp.float32),
    )


def kernel(x, edge_index, mask, W, b):
    row1 = edge_index[0]
    col1 = edge_index[1]
    degp = _make_deg()(col1.reshape(_EC, _CH))
    yp, mp, dinv2 = _make_prescale()(x, mask, degp.T)
    t2, t3, t1p = _make_agg()(yp, mp, dinv2.reshape(_NPAD), row1, col1)
    return _make_final()(t2, t3, t1p.T, dinv2, W, b.reshape(1, _D))


# speedup vs baseline: 1.0012x; 1.0012x over previous
import jax
import jax.numpy as jnp
from jax import lax
from jax.experimental import pallas as pl
from jax.experimental.pallas import tpu as pltpu
from jax.experimental.pallas import tpu_sc as plsc

_N = 10000
_E = 320000
_D = 128
_NPAD = 10240
_RPT = _NPAD // 16
_CH = 128
_EC = _E // _CH

_MESH = dict(core_axis_name="c", subcore_axis_name="s",
             num_cores=2, num_subcores=16)


_K1_CNT = 80


def _deg_body(col2_hbm, degp_hbm, degacc, stage, onesb, idxslab, sems):
    c = lax.axis_index("c")
    s = lax.axis_index("s")
    w = c * 16 + s

    def _z(i, _):
        stage[pl.ds(i * 16, 16)] = jnp.zeros((16,), jnp.float32)
        return 0

    lax.fori_loop(0, _RPT // 16, _z, 0)

    def _o(i, _):
        onesb[pl.ds(i * 16, 16)] = jnp.ones((16,), jnp.float32)
        return 0

    lax.fori_loop(0, _CH // 16, _o, 0)
    pltpu.sync_copy(stage, degacc.at[pl.ds(s * _RPT, _RPT)])

    @pl.when(w < 31)
    def _():
        pltpu.sync_copy(col2_hbm.at[pl.ds(w * _K1_CNT, _K1_CNT)], idxslab)

    @pl.when(w == 31)
    def _():
        pltpu.sync_copy(col2_hbm.at[pl.ds(31 * _K1_CNT, _EC - 31 * _K1_CNT)],
                        idxslab.at[pl.ds(0, _EC - 31 * _K1_CNT)])

    plsc.subcore_barrier()

    def _fire(j, _):
        pltpu.async_copy(onesb, degacc.at[idxslab.at[j]], sems, add=True)
        return 0

    def _drain(j, _):
        pltpu.make_async_copy(onesb, degacc.at[idxslab.at[0]], sems).wait()
        return 0

    @pl.when(w < 31)
    def _():
        lax.fori_loop(0, _K1_CNT, _fire, 0)
        lax.fori_loop(0, _K1_CNT, _drain, 0)

    @pl.when(w == 31)
    def _():
        lax.fori_loop(0, _EC - 31 * _K1_CNT, _fire, 0)
        lax.fori_loop(0, _EC - 31 * _K1_CNT, _drain, 0)

    plsc.subcore_barrier()
    pltpu.sync_copy(degacc.at[pl.ds(s * _RPT, _RPT)], stage)
    pltpu.sync_copy(stage, degp_hbm.at[c, pl.ds(s * _RPT, _RPT)])


def _make_deg():
    return pl.kernel(
        _deg_body,
        out_type=jax.ShapeDtypeStruct((2, _NPAD), jnp.float32),
        mesh=plsc.VectorSubcoreMesh(**_MESH),
        scratch_types=[
            pltpu.VMEM_SHARED((_NPAD,), jnp.float32),
            pltpu.VMEM((_RPT,), jnp.float32),
            pltpu.VMEM((_CH,), jnp.float32),
            pltpu.VMEM((_K1_CNT, _CH), jnp.int32),
            pltpu.SemaphoreType.DMA,
        ],
    )


_BLK = 1024


def _prescale_body(x_ref, m_ref, degt_ref, yp_ref, mp_ref, dinv_ref):
    dsum = degt_ref[:, 0:1] + degt_ref[:, 1:2]
    dv = jnp.where(dsum > 0, lax.rsqrt(dsum), 0.0)
    mm = m_ref[...]
    yp_ref[...] = x_ref[...] * mm * dv
    mp_ref[...] = mm * dv
    dinv_ref[...] = dv


def _make_prescale():
    return pl.pallas_call(
        _prescale_body,
        grid=(_NPAD // _BLK,),
        in_specs=[
            pl.BlockSpec((_BLK, _D), lambda i: (i, 0)),
            pl.BlockSpec((_BLK, _D), lambda i: (i, 0)),
            pl.BlockSpec((_BLK, 2), lambda i: (i, 0)),
        ],
        out_specs=[
            pl.BlockSpec((_BLK, _D), lambda i: (i, 0)),
            pl.BlockSpec((_BLK, _D), lambda i: (i, 0)),
            pl.BlockSpec((_BLK, 1), lambda i: (i, 0)),
        ],
        out_shape=[
            jax.ShapeDtypeStruct((_NPAD, _D), jnp.float32),
            jax.ShapeDtypeStruct((_NPAD, _D), jnp.float32),
            jax.ShapeDtypeStruct((_NPAD, 1), jnp.float32),
        ],
    )


_K3_CNT = _EC // 16


def _agg_body(yp_hbm, mp_hbm, dinv_hbm, row1_hbm, col1_hbm,
              t2_hbm, t3_hbm, t1p_hbm,
              acc, t1acc, cb0, cb1, cb2, cb3, rb0, rb1, rb2, rb3,
              db0, db1, vb0, vb1,
              si0, si1, si2, si3, sg0, sg1, ss0, ss1, sv0, sv1, st0, st1):
    c = lax.axis_index("c")
    s = lax.axis_index("s")
    colb = (cb0, cb1, cb2, cb3)
    rowb = (rb0, rb1, rb2, rb3)
    datab = (db0, db1)
    valsb = (vb0, vb1)
    semi = (si0, si1, si2, si3)
    semg = (sg0, sg1)
    sems = (ss0, ss1)
    semv = (sv0, sv1)
    semt = (st0, st1)

    def _zd(t, _):
        db0[t // 8, pl.ds((t % 8) * 16, 16)] = jnp.zeros((16,), jnp.float32)
        return 0

    lax.fori_loop(0, _CH * (_D // 16), _zd, 0)

    def _zv(i, _):
        vb0[pl.ds(i * 16, 16)] = jnp.zeros((16,), jnp.float32)
        return 0

    lax.fori_loop(0, _CH // 16, _zv, 0)

    row0 = s * _RPT
    for q in range(_RPT // _CH):
        pltpu.sync_copy(db0, acc.at[pl.ds(row0 + q * _CH, _CH)])
        pltpu.sync_copy(vb0, t1acc.at[pl.ds(row0 + q * _CH, _CH)])
    plsc.subcore_barrier()

    ebase = s * _K3_CNT

    def _issue_i(j, a):
        off = (ebase + j) * _CH
        pltpu.async_copy(col1_hbm.at[pl.ds(off, _CH)], colb[a], semi[a])
        pltpu.async_copy(row1_hbm.at[pl.ds(off, _CH)], rowb[a], semi[a])

    def _wait_i(j, a):
        off = (ebase + j) * _CH
        pltpu.make_async_copy(col1_hbm.at[pl.ds(off, _CH)], colb[a],
                              semi[a]).wait()
        pltpu.make_async_copy(row1_hbm.at[pl.ds(off, _CH)], rowb[a],
                              semi[a]).wait()

    def _run(table, with_t1, cbase, cnt):
        def issue_g(k, a, p):
            pltpu.async_copy(table.at[colb[a]], datab[p], semg[p])
            if with_t1:
                pltpu.async_copy(dinv_hbm.at[colb[a]], valsb[p], semv[p])

        def wait_g(a, p):
            pltpu.make_async_copy(table.at[colb[a]], datab[p], semg[p]).wait()
            if with_t1:
                pltpu.make_async_copy(dinv_hbm.at[colb[a]], valsb[p],
                                      semv[p]).wait()

        def issue_s(a, p):
            pltpu.async_copy(datab[p], acc.at[rowb[a]], sems[p], add=True)
            if with_t1:
                pltpu.async_copy(valsb[p], t1acc.at[rowb[a]], semt[p],
                                 add=True)

        def wait_s(a, p):
            pltpu.make_async_copy(datab[p], acc.at[rowb[a]], sems[p]).wait()
            if with_t1:
                pltpu.make_async_copy(valsb[p], t1acc.at[rowb[a]],
                                      semt[p]).wait()

        _issue_i(cbase + 0, 0)
        _issue_i(cbase + 1, 1)
        _issue_i(cbase + 2, 2)
        _wait_i(cbase + 0, 0)
        issue_g(0, 0, 0)
        _wait_i(cbase + 1, 1)
        issue_g(1, 1, 1)

        def _step(k, a, p):
            a2 = (a + 2) % 4
            wait_g(a2, p)
            issue_s(a2, p)
            wait_s(a2, p)
            _issue_i(cbase + k + 1, (a + 1) % 4)
            _wait_i(cbase + k, a)
            issue_g(k, a, p)

        def _quad(jj, _):
            k0 = 4 * jj + 2
            _step(k0, 2, 0)
            _step(k0 + 1, 3, 1)
            _step(k0 + 2, 0, 0)
            _step(k0 + 3, 1, 1)
            return 0

        def _tail_step(k, a, p):
            a2 = (a + 2) % 4
            wait_g(a2, p)
            issue_s(a2, p)
            wait_s(a2, p)
            _wait_i(cbase + k, a)
            issue_g(k, a, p)

        if (cnt - 2) % 4 == 2:
            lax.fori_loop(0, (cnt - 2) // 4, _quad, 0)
            k0 = cnt - 2
            _issue_i(cbase + cnt - 1, (cnt - 1) % 4)
            _tail_step(k0, k0 % 4, 0)
            _tail_step(k0 + 1, (k0 + 1) % 4, 1)
        else:
            lax.fori_loop(0, (cnt - 2) // 4 - 1, _quad, 0)
            k0 = cnt - 4
            _step(k0, k0 % 4, 0)
            _step(k0 + 1, (k0 + 1) % 4, 1)
            _step(k0 + 2, (k0 + 2) % 4, 0)
            _tail_step(k0 + 3, (k0 + 3) % 4, 1)
        wait_g((cnt - 2) % 4, 0)
        issue_s((cnt - 2) % 4, 0)
        wait_s((cnt - 2) % 4, 0)
        wait_g((cnt - 1) % 4, 1)
        issue_s((cnt - 1) % 4, 1)
        wait_s((cnt - 1) % 4, 1)

    def _run_core(table, first_half_t1, cnt):
        half = cnt // 2
        _run(table, first_half_t1, 0, half)
        _run(table, not first_half_t1, half, half)

    @pl.when(jnp.logical_and(c == 0, s < 15))
    def _():
        _run_core(yp_hbm, True, _K3_CNT)

    @pl.when(jnp.logical_and(c == 0, s == 15))
    def _():
        _run_core(yp_hbm, True, _EC - 15 * _K3_CNT)

    @pl.when(jnp.logical_and(c == 1, s < 15))
    def _():
        _run_core(mp_hbm, False, _K3_CNT)

    @pl.when(jnp.logical_and(c == 1, s == 15))
    def _():
        _run_core(mp_hbm, False, _EC - 15 * _K3_CNT)

    plsc.subcore_barrier()
    for q in range(_RPT // _CH):
        r = row0 + q * _CH

        @pl.when(c == 0)
        def _out0():
            pltpu.sync_copy(acc.at[pl.ds(r, _CH)], db0)
            pltpu.sync_copy(db0, t2_hbm.at[pl.ds(r, _CH)])

        @pl.when(c == 1)
        def _out1():
            pltpu.sync_copy(acc.at[pl.ds(r, _CH)], db0)
            pltpu.sync_copy(db0, t3_hbm.at[pl.ds(r, _CH)])

        pltpu.sync_copy(t1acc.at[pl.ds(r, _CH)], vb0)
        pltpu.sync_copy(vb0, t1p_hbm.at[c, pl.ds(r, _CH)])


def _make_agg():
    return pl.kernel(
        _agg_body,
        out_type=(
            jax.ShapeDtypeStruct((_NPAD, _D), jnp.float32),
            jax.ShapeDtypeStruct((_NPAD, _D), jnp.float32),
            jax.ShapeDtypeStruct((2, _NPAD), jnp.float32),
        ),
        mesh=plsc.VectorSubcoreMesh(**_MESH),
        scratch_types=[
            pltpu.VMEM_SHARED((_NPAD, _D), jnp.float32),
            pltpu.VMEM_SHARED((_NPAD,), jnp.float32),
            pltpu.VMEM((_CH,), jnp.int32),
            pltpu.VMEM((_CH,), jnp.int32),
            pltpu.VMEM((_CH,), jnp.int32),
            pltpu.VMEM((_CH,), jnp.int32),
            pltpu.VMEM((_CH,), jnp.int32),
            pltpu.VMEM((_CH,), jnp.int32),
            pltpu.VMEM((_CH,), jnp.int32),
            pltpu.VMEM((_CH,), jnp.int32),
            pltpu.VMEM((_CH, _D), jnp.float32),
            pltpu.VMEM((_CH, _D), jnp.float32),
            pltpu.VMEM((_CH,), jnp.float32),
            pltpu.VMEM((_CH,), jnp.float32),
            pltpu.SemaphoreType.DMA,
            pltpu.SemaphoreType.DMA,
            pltpu.SemaphoreType.DMA,
            pltpu.SemaphoreType.DMA,
            pltpu.SemaphoreType.DMA,
            pltpu.SemaphoreType.DMA,
            pltpu.SemaphoreType.DMA,
            pltpu.SemaphoreType.DMA,
            pltpu.SemaphoreType.DMA,
            pltpu.SemaphoreType.DMA,
            pltpu.SemaphoreType.DMA,
            pltpu.SemaphoreType.DMA,
        ],
    )


def _final_body(t2_ref, t3_ref, t1p_ref, dinv_ref, w_ref, b_ref, o_ref):
    dv = dinv_ref[...]
    t1 = t1p_ref[:, 0:1] + t1p_ref[:, 1:2]
    t3 = t3_ref[...]
    safe = jnp.where(t3 != 0, t3, 1.0)
    nz = (t3 != 0) & (dv != 0)
    ratio = jnp.where(nz, dv * t1 * t2_ref[...] / safe, 0.0)
    o_ref[...] = lax.dot_general(
        ratio, w_ref[...], (((1,), (1,)), ((), ())),
        preferred_element_type=jnp.float32) + b_ref[...]


def _make_final():
    return pl.pallas_call(
        _final_body,
        grid=(_NPAD // _BLK,),
        in_specs=[
            pl.BlockSpec((_BLK, _D), lambda i: (i, 0)),
            pl.BlockSpec((_BLK, _D), lambda i: (i, 0)),
            pl.BlockSpec((_BLK, 2), lambda i: (i, 0)),
            pl.BlockSpec((_BLK, 1), lambda i: (i, 0)),
            pl.BlockSpec((_D, _D), lambda i: (0, 0)),
            pl.BlockSpec((1, _D), lambda i: (0, 0)),
        ],
        out_specs=pl.BlockSpec((_BLK, _D), lambda i: (i, 0)),
        out_shape=jax.ShapeDtypeStruct((_N, _D), jnp.float32),
    )


def kernel(x, edge_index, mask, W, b):
    row1 = edge_index[0]
    col1 = edge_index[1]
    degp = _make_deg()(col1.reshape(_EC, _CH))
    yp, mp, dinv2 = _make_prescale()(x, mask, degp.T)
    t2, t3, t1p = _make_agg()(yp, mp, dinv2.reshape(_NPAD), row1, col1)
    return _make_final()(t2, t3, t1p.T, dinv2, W, b.reshape(1, _D))
